# baseline (speedup 1.0000x reference)
import functools

import jax
import jax.numpy as jnp
import numpy as np
from jax.experimental import pallas as pl
from jax.experimental.pallas import tpu as pltpu

_H = 32
_HEADS = 8
_NHEAD = 4


def _seg_softmax(alpha, index, num):
    amax = jax.ops.segment_max(alpha, index, num_segments=num)
    amax = jnp.where(jnp.isfinite(amax), amax, 0.0)
    a = jnp.exp(alpha - amax[index])
    s = jax.ops.segment_sum(a, index, num_segments=num)
    return a / (s[index] + 1e-16)


def _gat(x, src, dst, ea, Wl, bl, Wr, br, att, bias, We, heads, oc, n):
    xl = (x @ Wl.T + bl).reshape(n, heads, oc)
    xr = (x @ Wr.T + br).reshape(n, heads, oc)
    xj = xl[src]
    m = xr[dst] + xj
    if ea is not None:
        m = m + (ea @ We.T).reshape(-1, heads, oc)
    m = jax.nn.leaky_relu(m, 0.2)
    alpha = (m * att[None]).sum(-1)
    alpha = _seg_softmax(alpha, dst, n)
    out = jax.ops.segment_sum(xj * alpha[..., None], dst, n)
    return out.reshape(n, heads * oc) + bias


def _tail_body(y_ref, z_ref, wm_ref, wv_ref, o_ref):
    y = y_ref[...]
    z = z_ref[...]

    def mm(t, i):
        return jnp.dot(t, wm_ref[i], preferred_element_type=jnp.float32)

    def bias(t, i):
        return t + wv_ref[i][None, :]

    def ln(t, gi, bi):
        mu = jnp.mean(t, axis=-1, keepdims=True)
        var = jnp.mean((t - mu) ** 2, axis=-1, keepdims=True)
        return (t - mu) * jax.lax.rsqrt(var + 1e-5) * wv_ref[gi][None, :] + wv_ref[bi][None, :]

    mem = y
    t = y
    for l in range(2):
        mi, vi = 6 * l, 12 * l
        h = bias(mm(bias(mm(t, mi + 0), vi + 0), mi + 1), vi + 1)
        t = ln(t + h, vi + 6, vi + 7)
        h = bias(mm(bias(mm(mem, mi + 2), vi + 2), mi + 3), vi + 3)
        t = ln(t + h, vi + 8, vi + 9)
        h = bias(mm(jnp.maximum(bias(mm(t, mi + 4), vi + 4), 0.0), mi + 5), vi + 5)
        t = ln(t + h, vi + 10, vi + 11)
    t = jnp.tanh(t)
    t = bias(mm(t, 12), 24) + z
    t = jnp.where(t > 0, t, 0.01 * t)
    t = bias(mm(t, 13), 25)
    t = jnp.where(t > 0, t, 0.01 * t)
    t = bias(mm(t, 14), 26)
    head = t
    head = jnp.where(head < 0, head + 1, head)
    head = jnp.where(head > 1, head - 1, head)
    col = jax.lax.broadcasted_iota(jnp.int32, t.shape, 1)
    o_ref[...] = jnp.where(col < 2, head, t)


def _run_tail(y, z, wmats, wvecs, n):
    bn = 1000
    return pl.pallas_call(
        _tail_body,
        grid=(n // bn,),
        in_specs=[
            pl.BlockSpec((bn, _H), lambda i: (i, 0)),
            pl.BlockSpec((bn, _H), lambda i: (i, 0)),
            pl.BlockSpec((15, _H, _H), lambda i: (0, 0, 0)),
            pl.BlockSpec((27, _H), lambda i: (0, 0)),
        ],
        out_specs=pl.BlockSpec((bn, _H), lambda i: (i, 0)),
        out_shape=jax.ShapeDtypeStruct((n, _H), jnp.float32),
    )(y, z, wmats, wvecs)


def kernel(x, edge_index, edge_attr, params, weights):
    xshape = x.shape
    n = xshape[0] * xshape[1]
    c = xshape[2]
    ne = edge_index.shape[1]
    w = weights

    ea = edge_attr.reshape(ne, -1)
    pe = jnp.broadcast_to(params.reshape(1, 4), (ne, 4))
    xf = x.reshape(n, c)
    ea = jnp.concatenate([ea, pe, xf[edge_index[0]], xf[edge_index[1]]], axis=-1)
    loop = jnp.arange(n, dtype=edge_index.dtype)
    src = jnp.concatenate([edge_index[0], loop])
    dst = jnp.concatenate([edge_index[1], loop])
    ea_full = jnp.concatenate(
        [ea, jnp.broadcast_to(ea.mean(axis=0, keepdims=True), (n, ea.shape[1]))], axis=0)

    y = jax.nn.elu(_gat(xf, src, dst, ea_full, w['g0_Wl'], w['g0_bl'], w['g0_Wr'],
                        w['g0_br'], w['g0_att'], w['g0_bias'], w['g0_We'], _HEADS, _H, n))
    y = jax.nn.elu(_gat(y, src, dst, None, w['g1_Wl'], w['g1_bl'], w['g1_Wr'],
                        w['g1_br'], w['g1_att'], w['g1_bias'], None, _HEADS, _H, n))
    y = jax.nn.elu(_gat(y, src, dst, None, w['g2_Wl'], w['g2_bl'], w['g2_Wr'],
                        w['g2_br'], w['g2_att'], w['g2_bias'], None, 1, _H, n))

    z = jnp.concatenate([xf, jnp.zeros((n, _H - c), jnp.float32)], axis=1)
    y = y + z

    mats, vecs = [], []
    for l in range(2):
        p = 't%d' % l
        mats += [w[p + '_saW'][2 * _H:].T, w[p + '_saoW'].T,
                 w[p + '_caW'][2 * _H:].T, w[p + '_caoW'].T,
                 w[p + '_l1W'].T, w[p + '_l2W'].T]
        vecs += [w[p + '_sab'][2 * _H:], w[p + '_saob'], w[p + '_cab'][2 * _H:],
                 w[p + '_caob'], w[p + '_l1b'], w[p + '_l2b'],
                 w[p + '_n1g'], w[p + '_n1b'], w[p + '_n2g'], w[p + '_n2b'],
                 w[p + '_n3g'], w[p + '_n3b']]
    dr2W = jnp.zeros((_H, _H), jnp.float32).at[:4].set(w['dr2_W'])
    dr2b = jnp.zeros((_H,), jnp.float32).at[:4].set(w['dr2_b'])
    mats += [w['dr1_W'].T, w['dr11_W'].T, dr2W.T]
    vecs += [w['dr1_b'], w['dr11_b'], dr2b]
    wmats = jnp.stack(mats)
    wvecs = jnp.stack(vecs)

    out = _run_tail(y, z, wmats, wvecs, n)
    return out[:, :c].reshape(xshape)
